# baseline (speedup 1.0000x reference)
import functools

import jax
import jax.numpy as jnp
from jax import lax
from jax.experimental import pallas as pl
from jax.experimental.pallas import tpu as pltpu
from jax.experimental.pallas import tpu_sc as plsc

_N = 10000
_NPAD = 10112
_E = 320000
_D1 = 128
_DH = _D1 // 2
_D2 = 16
_C = 128
_NCORES = 2
_NSUB = 16
_ROWS_PER_TILE = _NPAD // _NSUB
_XROWS_PER_TILE = _N // _NSUB

_ROWS_TOTAL = _E // _C
_NCH2 = 78
_PHASES1 = 2
_TAIL = _ROWS_TOTAL - 32 * _NCH2


def _pipeline(table, srcv, dstv, acc, buf0, buf1, sem0, sem1, n_chunks):
  n_pairs = n_chunks // 2
  pltpu.async_copy(table.at[srcv.at[0]], buf0, sem0)

  def body(i, carry):
    j0 = 2 * i
    d1 = pltpu.async_copy(table.at[srcv.at[j0 + 1]], buf1, sem1)
    pltpu.make_async_copy(table.at[srcv.at[j0]], buf0, sem0).wait()
    pltpu.sync_copy(buf0, acc.at[dstv.at[j0]], add=True)
    jpre = jnp.minimum(j0 + 2, n_chunks - 1)
    pltpu.async_copy(table.at[srcv.at[jpre]], buf0, sem0)
    d1.wait()
    pltpu.sync_copy(buf1, acc.at[dstv.at[j0 + 1]], add=True)
    return carry

  lax.fori_loop(0, n_pairs, body, 0)
  pltpu.make_async_copy(table.at[srcv.at[n_chunks - 1]], buf0,
                        sem0).wait()


def _pipeline3(table, thbm, srcv, dstv, acc, buf0, buf1, buf2,
               sem0, sem1, sem2, n_chunks):
  n_groups = n_chunks // 3
  pltpu.async_copy(table.at[srcv.at[0]], buf0, sem0)

  def body(g, carry):
    j0 = 3 * g
    d2 = pltpu.async_copy(thbm.at[srcv.at[j0 + 2]], buf2, sem2)
    d1 = pltpu.async_copy(table.at[srcv.at[j0 + 1]], buf1, sem1)
    pltpu.make_async_copy(table.at[srcv.at[j0]], buf0, sem0).wait()
    pltpu.sync_copy(buf0, acc.at[dstv.at[j0]], add=True)
    jpre = jnp.minimum(j0 + 3, n_chunks - 1)
    pltpu.async_copy(table.at[srcv.at[jpre]], buf0, sem0)
    d1.wait()
    pltpu.sync_copy(buf1, acc.at[dstv.at[j0 + 1]], add=True)
    d2.wait()
    pltpu.sync_copy(buf2, acc.at[dstv.at[j0 + 2]], add=True)
    return carry

  lax.fori_loop(0, n_groups, body, 0)
  pltpu.make_async_copy(table.at[srcv.at[n_chunks - 1]], buf0,
                        sem0).wait()


def _tail_chunk(table, src_hbm, dst_hbm, row, srcv, dstv, acc, buf0,
                sem0):
  pltpu.sync_copy(src_hbm.at[pl.ds(row, 1)], srcv.at[pl.ds(0, 1)])
  pltpu.sync_copy(dst_hbm.at[pl.ds(row, 1)], dstv.at[pl.ds(0, 1)])
  pltpu.async_copy(table.at[srcv.at[0]], buf0, sem0).wait()
  pltpu.sync_copy(buf0, acc.at[dstv.at[0]], add=True)


def _agg1_body(x_hbm, src_hbm, dst_hbm, zeros_hbm, out_hbm,
               xsh, acc, srcv, dstv, buf0, buf1, buf2, sem0, sem1, sem2):
  c = lax.axis_index("c")
  s = lax.axis_index("s")
  r0 = s * _ROWS_PER_TILE
  pltpu.sync_copy(
      x_hbm.at[c, pl.ds(s * _XROWS_PER_TILE, _XROWS_PER_TILE)],
      xsh.at[pl.ds(s * _XROWS_PER_TILE, _XROWS_PER_TILE)])
  pltpu.sync_copy(zeros_hbm.at[pl.ds(r0, _ROWS_PER_TILE)],
                  acc.at[pl.ds(r0, _ROWS_PER_TILE)])
  plsc.subcore_barrier()
  for p in range(_PHASES1):
    row0 = s * (_PHASES1 * _NCH2) + p * _NCH2
    pltpu.sync_copy(src_hbm.at[pl.ds(row0, _NCH2)], srcv)
    pltpu.sync_copy(dst_hbm.at[pl.ds(row0, _NCH2)], dstv)
    _pipeline3(xsh, x_hbm.at[c], srcv, dstv, acc, buf0, buf1, buf2,
               sem0, sem1, sem2, _NCH2)

  @pl.when(s < _TAIL)
  def _():
    _tail_chunk(xsh, src_hbm, dst_hbm, 32 * _NCH2 + s, srcv, dstv, acc,
                buf0, sem0)

  plsc.subcore_barrier()
  pltpu.sync_copy(acc.at[pl.ds(r0, _ROWS_PER_TILE)],
                  out_hbm.at[c, pl.ds(r0, _ROWS_PER_TILE)])


def _agg2_body(h_hbm, src_hbm, dst_hbm, zeros_hbm, out_hbm,
               hsh, acc, srcv, dstv, buf0, buf1, sem0, sem1):
  c = lax.axis_index("c")
  s = lax.axis_index("s")
  r0 = s * _ROWS_PER_TILE
  pltpu.sync_copy(h_hbm.at[pl.ds(r0, _ROWS_PER_TILE)],
                  hsh.at[pl.ds(r0, _ROWS_PER_TILE)])
  pltpu.sync_copy(zeros_hbm.at[pl.ds(r0, _ROWS_PER_TILE)],
                  acc.at[pl.ds(r0, _ROWS_PER_TILE)])
  t = c * _NSUB + s
  row0 = t * _NCH2
  pltpu.sync_copy(src_hbm.at[pl.ds(row0, _NCH2)], srcv)
  pltpu.sync_copy(dst_hbm.at[pl.ds(row0, _NCH2)], dstv)
  plsc.subcore_barrier()
  _pipeline(hsh, srcv, dstv, acc, buf0, buf1, sem0, sem1, _NCH2)

  @pl.when(t < _TAIL)
  def _():
    _tail_chunk(hsh, src_hbm, dst_hbm, 32 * _NCH2 + t, srcv, dstv, acc,
                buf0, sem0)

  plsc.subcore_barrier()
  pltpu.sync_copy(acc.at[pl.ds(r0, _ROWS_PER_TILE)],
                  out_hbm.at[c, pl.ds(r0, _ROWS_PER_TILE)])


@functools.cache
def _build_aggs():
  mesh = plsc.VectorSubcoreMesh(
      core_axis_name="c", subcore_axis_name="s",
      num_cores=_NCORES, num_subcores=_NSUB)
  params = pltpu.CompilerParams(use_tc_tiling_on_sc=False)
  agg1 = pl.kernel(
      _agg1_body,
      out_type=jax.ShapeDtypeStruct((_NCORES, _NPAD, _DH), jnp.float32),
      mesh=mesh,
      scratch_types=[
          pltpu.VMEM_SHARED((_NPAD, _DH), jnp.float32),
          pltpu.VMEM_SHARED((_NPAD, _DH), jnp.float32),
          pltpu.VMEM((_NCH2, _C), jnp.int32),
          pltpu.VMEM((_NCH2, _C), jnp.int32),
          pltpu.VMEM((_C, _DH), jnp.float32),
          pltpu.VMEM((_C, _DH), jnp.float32),
          pltpu.VMEM((_C, _DH), jnp.float32),
          pltpu.SemaphoreType.DMA,
          pltpu.SemaphoreType.DMA,
          pltpu.SemaphoreType.DMA,
      ],
      compiler_params=params)
  agg2 = pl.kernel(
      _agg2_body,
      out_type=jax.ShapeDtypeStruct((_NCORES, _NPAD, _D2), jnp.float32),
      mesh=mesh,
      scratch_types=[
          pltpu.VMEM_SHARED((_NPAD, _D2), jnp.float32),
          pltpu.VMEM_SHARED((_NPAD, _D2), jnp.float32),
          pltpu.VMEM((_NCH2, _C), jnp.int32),
          pltpu.VMEM((_NCH2, _C), jnp.int32),
          pltpu.VMEM((_C, _D2), jnp.float32),
          pltpu.VMEM((_C, _D2), jnp.float32),
          pltpu.SemaphoreType.DMA,
          pltpu.SemaphoreType.DMA,
      ],
      compiler_params=params)
  return agg1, agg2


def _mm_body(a_ref, w1_ref, w2_ref, o_ref):
  wc = jnp.dot(w1_ref[...], w2_ref[...],
               preferred_element_type=jnp.float32)
  o_ref[...] = (
      jnp.dot(a_ref[0], wc[:_DH], preferred_element_type=jnp.float32)
      + jnp.dot(a_ref[1], wc[_DH:], preferred_element_type=jnp.float32))


_mm = pl.pallas_call(
    _mm_body, out_shape=jax.ShapeDtypeStruct((_NPAD, _D2), jnp.float32))


def _add_body(a_ref, b_ref, o_ref):
  o_ref[...] = a_ref[:_N] + b_ref[:_N]


_add = pl.pallas_call(
    _add_body, out_shape=jax.ShapeDtypeStruct((_N, _D2), jnp.float32))


@jax.jit
def kernel(x, edge_index, W1, W2):
  src3 = edge_index[0].astype(jnp.int32).reshape(_ROWS_TOTAL, _C)
  dst3 = edge_index[1].astype(jnp.int32).reshape(_ROWS_TOTAL, _C)
  x_split = jnp.stack([x[:, :_DH], x[:, _DH:]])

  _agg1, _agg2 = _build_aggs()
  zeros1 = jnp.zeros((_NPAD, _DH), jnp.float32)
  agg1 = _agg1(x_split, src3, dst3, zeros1)

  h2 = _mm(agg1, W1, W2)

  zeros2 = jnp.zeros((_NPAD, _D2), jnp.float32)
  parts = _agg2(h2, src3, dst3, zeros2)

  return _add(parts[0], parts[1])


# speedup vs baseline: 1.1369x; 1.1369x over previous
import functools

import jax
import jax.numpy as jnp
from jax import lax
from jax.experimental import pallas as pl
from jax.experimental.pallas import tpu as pltpu
from jax.experimental.pallas import tpu_sc as plsc

_N = 10000
_NPAD = 10112
_E = 320000
_D1 = 128
_DH = _D1 // 2
_D2 = 16
_C = 128
_NCORES = 2
_NSUB = 16
_ROWS_PER_TILE = _NPAD // _NSUB
_XROWS_PER_TILE = _N // _NSUB

_ROWS_TOTAL = _E // _C
_NCH2 = 78
_PHASES1 = 2
_TAIL = _ROWS_TOTAL - 32 * _NCH2


def _pipeline(table, srcv, dstv, acc, buf0, buf1, sem0, sem1, n_chunks):
  n_pairs = n_chunks // 2
  pltpu.async_copy(table.at[srcv.at[0]], buf0, sem0)

  def body(i, carry):
    j0 = 2 * i
    d1 = pltpu.async_copy(table.at[srcv.at[j0 + 1]], buf1, sem1)
    pltpu.make_async_copy(table.at[srcv.at[j0]], buf0, sem0).wait()
    pltpu.sync_copy(buf0, acc.at[dstv.at[j0]], add=True)
    jpre = jnp.minimum(j0 + 2, n_chunks - 1)
    pltpu.async_copy(table.at[srcv.at[jpre]], buf0, sem0)
    d1.wait()
    pltpu.sync_copy(buf1, acc.at[dstv.at[j0 + 1]], add=True)
    return carry

  lax.fori_loop(0, n_pairs, body, 0)
  pltpu.make_async_copy(table.at[srcv.at[n_chunks - 1]], buf0,
                        sem0).wait()


def _pipeline3(table, thbm, srcv, dstv, acc, buf0, buf1, buf2,
               sem0, sem1, sem2, n_chunks):
  n_groups = n_chunks // 3
  pltpu.async_copy(table.at[srcv.at[0]], buf0, sem0)

  def body(g, carry):
    j0 = 3 * g
    d2 = pltpu.async_copy(thbm.at[srcv.at[j0 + 2]], buf2, sem2)
    d1 = pltpu.async_copy(table.at[srcv.at[j0 + 1]], buf1, sem1)
    pltpu.make_async_copy(table.at[srcv.at[j0]], buf0, sem0).wait()
    pltpu.sync_copy(buf0, acc.at[dstv.at[j0]], add=True)
    jpre = jnp.minimum(j0 + 3, n_chunks - 1)
    pltpu.async_copy(table.at[srcv.at[jpre]], buf0, sem0)
    d1.wait()
    pltpu.sync_copy(buf1, acc.at[dstv.at[j0 + 1]], add=True)
    d2.wait()
    pltpu.sync_copy(buf2, acc.at[dstv.at[j0 + 2]], add=True)
    return carry

  lax.fori_loop(0, n_groups, body, 0)
  pltpu.make_async_copy(table.at[srcv.at[n_chunks - 1]], buf0,
                        sem0).wait()


def _tail_chunk(table, src_hbm, dst_hbm, row, srcv, dstv, acc, buf0,
                sem0):
  pltpu.sync_copy(src_hbm.at[pl.ds(row, 1)], srcv.at[pl.ds(0, 1)])
  pltpu.sync_copy(dst_hbm.at[pl.ds(row, 1)], dstv.at[pl.ds(0, 1)])
  pltpu.async_copy(table.at[srcv.at[0]], buf0, sem0).wait()
  pltpu.sync_copy(buf0, acc.at[dstv.at[0]], add=True)


def _agg1_body(x_hbm, src_hbm, dst_hbm, zeros_hbm, out_hbm,
               xsh, acc, srcv, dstv, buf0, buf1, sem0, sem1):
  c = lax.axis_index("c")
  s = lax.axis_index("s")
  r0 = s * _ROWS_PER_TILE
  pltpu.sync_copy(
      x_hbm.at[pl.ds(s * _XROWS_PER_TILE, _XROWS_PER_TILE),
               pl.ds(c * _DH, _DH)],
      xsh.at[pl.ds(s * _XROWS_PER_TILE, _XROWS_PER_TILE)])
  pltpu.sync_copy(zeros_hbm.at[pl.ds(r0, _ROWS_PER_TILE)],
                  acc.at[pl.ds(r0, _ROWS_PER_TILE)])
  plsc.subcore_barrier()
  for p in range(_PHASES1):
    row0 = s * (_PHASES1 * _NCH2) + p * _NCH2
    pltpu.sync_copy(src_hbm.at[pl.ds(row0, _NCH2)], srcv)
    pltpu.sync_copy(dst_hbm.at[pl.ds(row0, _NCH2)], dstv)
    _pipeline(xsh, srcv, dstv, acc, buf0, buf1, sem0, sem1, _NCH2)

  @pl.when(s < _TAIL)
  def _():
    _tail_chunk(xsh, src_hbm, dst_hbm, 32 * _NCH2 + s, srcv, dstv, acc,
                buf0, sem0)

  plsc.subcore_barrier()
  pltpu.sync_copy(acc.at[pl.ds(r0, _ROWS_PER_TILE)],
                  out_hbm.at[c, pl.ds(r0, _ROWS_PER_TILE)])


def _agg2_body(h_hbm, src_hbm, dst_hbm, zeros_hbm, out_hbm,
               hsh, acc, srcv, dstv, buf0, buf1, sem0, sem1):
  c = lax.axis_index("c")
  s = lax.axis_index("s")
  r0 = s * _ROWS_PER_TILE
  pltpu.sync_copy(h_hbm.at[pl.ds(r0, _ROWS_PER_TILE)],
                  hsh.at[pl.ds(r0, _ROWS_PER_TILE)])
  pltpu.sync_copy(zeros_hbm.at[pl.ds(r0, _ROWS_PER_TILE)],
                  acc.at[pl.ds(r0, _ROWS_PER_TILE)])
  t = c * _NSUB + s
  row0 = t * _NCH2
  pltpu.sync_copy(src_hbm.at[pl.ds(row0, _NCH2)], srcv)
  pltpu.sync_copy(dst_hbm.at[pl.ds(row0, _NCH2)], dstv)
  plsc.subcore_barrier()
  _pipeline(hsh, srcv, dstv, acc, buf0, buf1, sem0, sem1, _NCH2)

  @pl.when(t < _TAIL)
  def _():
    _tail_chunk(hsh, src_hbm, dst_hbm, 32 * _NCH2 + t, srcv, dstv, acc,
                buf0, sem0)

  plsc.subcore_barrier()
  pltpu.sync_copy(acc.at[pl.ds(r0, _ROWS_PER_TILE)],
                  out_hbm.at[c, pl.ds(r0, _ROWS_PER_TILE)])


@functools.cache
def _build_aggs():
  mesh = plsc.VectorSubcoreMesh(
      core_axis_name="c", subcore_axis_name="s",
      num_cores=_NCORES, num_subcores=_NSUB)
  params = pltpu.CompilerParams(use_tc_tiling_on_sc=False)
  agg1 = pl.kernel(
      _agg1_body,
      out_type=jax.ShapeDtypeStruct((_NCORES, _NPAD, _DH), jnp.float32),
      mesh=mesh,
      scratch_types=[
          pltpu.VMEM_SHARED((_NPAD, _DH), jnp.float32),
          pltpu.VMEM_SHARED((_NPAD, _DH), jnp.float32),
          pltpu.VMEM((_NCH2, _C), jnp.int32),
          pltpu.VMEM((_NCH2, _C), jnp.int32),
          pltpu.VMEM((_C, _DH), jnp.float32),
          pltpu.VMEM((_C, _DH), jnp.float32),
          pltpu.SemaphoreType.DMA,
          pltpu.SemaphoreType.DMA,
      ],
      compiler_params=params)
  agg2 = pl.kernel(
      _agg2_body,
      out_type=jax.ShapeDtypeStruct((_NCORES, _NPAD, _D2), jnp.float32),
      mesh=mesh,
      scratch_types=[
          pltpu.VMEM_SHARED((_NPAD, _D2), jnp.float32),
          pltpu.VMEM_SHARED((_NPAD, _D2), jnp.float32),
          pltpu.VMEM((_NCH2, _C), jnp.int32),
          pltpu.VMEM((_NCH2, _C), jnp.int32),
          pltpu.VMEM((_C, _D2), jnp.float32),
          pltpu.VMEM((_C, _D2), jnp.float32),
          pltpu.SemaphoreType.DMA,
          pltpu.SemaphoreType.DMA,
      ],
      compiler_params=params)
  return agg1, agg2


def _mm_body(a_ref, w1_ref, w2_ref, o_ref):
  wc = jnp.dot(w1_ref[...], w2_ref[...],
               preferred_element_type=jnp.float32)
  rows = lax.broadcasted_iota(jnp.int32, (8 * _DH, 8 * _D2), 0) // _DH
  cols = lax.broadcasted_iota(jnp.int32, (8 * _DH, 8 * _D2), 1) // _D2
  on_diag = rows == cols
  bda = jnp.where(on_diag, jnp.tile(wc[:_DH], (8, 8)), 0.0)
  bdb = jnp.where(on_diag, jnp.tile(wc[_DH:], (8, 8)), 0.0)
  o_ref[...] = (
      jnp.dot(a_ref[0], bda, preferred_element_type=jnp.float32)
      + jnp.dot(a_ref[1], bdb, preferred_element_type=jnp.float32))


_mm = pl.pallas_call(
    _mm_body,
    out_shape=jax.ShapeDtypeStruct((_NPAD // 8, 8 * _D2), jnp.float32))


def _add_body(a_ref, b_ref, o_ref):
  o_ref[...] = a_ref[:_N // 8] + b_ref[:_N // 8]


_add = pl.pallas_call(
    _add_body,
    out_shape=jax.ShapeDtypeStruct((_N // 8, 8 * _D2), jnp.float32))


@jax.jit
def kernel(x, edge_index, W1, W2):
  src3 = edge_index[0].astype(jnp.int32).reshape(_ROWS_TOTAL, _C)
  dst3 = edge_index[1].astype(jnp.int32).reshape(_ROWS_TOTAL, _C)

  _agg1, _agg2 = _build_aggs()
  zeros1 = jnp.zeros((_NPAD, _DH), jnp.float32)
  agg1 = _agg1(x, src3, dst3, zeros1)

  h2 = _mm(agg1.reshape(_NCORES, _NPAD // 8, 8 * _DH), W1, W2)

  zeros2 = jnp.zeros((_NPAD, _D2), jnp.float32)
  parts = _agg2(h2.reshape(_NPAD, _D2), src3, dst3, zeros2)

  out = _add(parts[0].reshape(_NPAD // 8, 8 * _D2),
             parts[1].reshape(_NPAD // 8, 8 * _D2))
  return out.reshape(_N, _D2)


# speedup vs baseline: 1.2954x; 1.1393x over previous
import functools

import jax
import jax.numpy as jnp
from jax import lax
from jax.experimental import pallas as pl
from jax.experimental.pallas import tpu as pltpu
from jax.experimental.pallas import tpu_sc as plsc

_N = 10000
_NPAD = 10112
_E = 320000
_D1 = 128
_DH = _D1 // 2
_D2 = 16
_C = 128
_NCORES = 2
_NSUB = 16
_ROWS_PER_TILE = _NPAD // _NSUB
_XROWS_PER_TILE = _N // _NSUB

_ROWS_TOTAL = _E // _C
_NCH2 = 78
_PHASES1 = 2
_TAIL = _ROWS_TOTAL - 32 * _NCH2


def _pipeline(table, srcv, dstv, acc, buf0, buf1, sem0, sem1, n_chunks):
  n_pairs = n_chunks // 2
  pltpu.async_copy(table.at[srcv.at[0]], buf0, sem0)

  def body(i, carry):
    j0 = 2 * i
    d1 = pltpu.async_copy(table.at[srcv.at[j0 + 1]], buf1, sem1)
    pltpu.make_async_copy(table.at[srcv.at[j0]], buf0, sem0).wait()
    pltpu.sync_copy(buf0, acc.at[dstv.at[j0]], add=True)
    jpre = jnp.minimum(j0 + 2, n_chunks - 1)
    pltpu.async_copy(table.at[srcv.at[jpre]], buf0, sem0)
    d1.wait()
    pltpu.sync_copy(buf1, acc.at[dstv.at[j0 + 1]], add=True)
    return carry

  lax.fori_loop(0, n_pairs, body, 0)
  pltpu.make_async_copy(table.at[srcv.at[n_chunks - 1]], buf0,
                        sem0).wait()


def _pipeline3(table, thbm, srcv, dstv, acc, buf0, buf1, buf2,
               sem0, sem1, sem2, n_chunks):
  n_groups = n_chunks // 3
  pltpu.async_copy(table.at[srcv.at[0]], buf0, sem0)

  def body(g, carry):
    j0 = 3 * g
    d2 = pltpu.async_copy(thbm.at[srcv.at[j0 + 2]], buf2, sem2)
    d1 = pltpu.async_copy(table.at[srcv.at[j0 + 1]], buf1, sem1)
    pltpu.make_async_copy(table.at[srcv.at[j0]], buf0, sem0).wait()
    pltpu.sync_copy(buf0, acc.at[dstv.at[j0]], add=True)
    jpre = jnp.minimum(j0 + 3, n_chunks - 1)
    pltpu.async_copy(table.at[srcv.at[jpre]], buf0, sem0)
    d1.wait()
    pltpu.sync_copy(buf1, acc.at[dstv.at[j0 + 1]], add=True)
    d2.wait()
    pltpu.sync_copy(buf2, acc.at[dstv.at[j0 + 2]], add=True)
    return carry

  lax.fori_loop(0, n_groups, body, 0)
  pltpu.make_async_copy(table.at[srcv.at[n_chunks - 1]], buf0,
                        sem0).wait()


def _tail_chunk(table, ei_hbm, row, srcv, dstv, acc, buf0, sem0):
  pltpu.sync_copy(ei_hbm.at[0, pl.ds(row, 1)], srcv.at[pl.ds(0, 1)])
  pltpu.sync_copy(ei_hbm.at[1, pl.ds(row, 1)], dstv.at[pl.ds(0, 1)])
  pltpu.async_copy(table.at[srcv.at[0]], buf0, sem0).wait()
  pltpu.sync_copy(buf0, acc.at[dstv.at[0]], add=True)


def _agg1_body(x_hbm, ei_hbm, zeros_hbm, out_hbm,
               xsh, acc, srcv, dstv, buf0, buf1, sem0, sem1):
  c = lax.axis_index("c")
  s = lax.axis_index("s")
  r0 = s * _ROWS_PER_TILE
  pltpu.sync_copy(
      x_hbm.at[pl.ds(s * _XROWS_PER_TILE, _XROWS_PER_TILE),
               pl.ds(c * _DH, _DH)],
      xsh.at[pl.ds(s * _XROWS_PER_TILE, _XROWS_PER_TILE)])
  pltpu.sync_copy(zeros_hbm.at[pl.ds(r0, _ROWS_PER_TILE)],
                  acc.at[pl.ds(r0, _ROWS_PER_TILE)])
  plsc.subcore_barrier()
  for p in range(_PHASES1):
    row0 = s * (_PHASES1 * _NCH2) + p * _NCH2
    pltpu.sync_copy(ei_hbm.at[0, pl.ds(row0, _NCH2)], srcv)
    pltpu.sync_copy(ei_hbm.at[1, pl.ds(row0, _NCH2)], dstv)
    _pipeline(xsh, srcv, dstv, acc, buf0, buf1, sem0, sem1, _NCH2)

  @pl.when(s < _TAIL)
  def _():
    _tail_chunk(xsh, ei_hbm, 32 * _NCH2 + s, srcv, dstv, acc, buf0,
                sem0)

  plsc.subcore_barrier()
  pltpu.sync_copy(acc.at[pl.ds(r0, _ROWS_PER_TILE)],
                  out_hbm.at[c, pl.ds(r0, _ROWS_PER_TILE)])


def _agg2_body(h_hbm, ei_hbm, zeros_hbm, out_hbm,
               hsh, acc, srcv, dstv, buf0, buf1, sem0, sem1):
  c = lax.axis_index("c")
  s = lax.axis_index("s")
  r0 = s * _ROWS_PER_TILE
  pltpu.sync_copy(h_hbm.at[pl.ds(r0, _ROWS_PER_TILE)],
                  hsh.at[pl.ds(r0, _ROWS_PER_TILE)])
  pltpu.sync_copy(zeros_hbm.at[pl.ds(r0, _ROWS_PER_TILE)],
                  acc.at[pl.ds(r0, _ROWS_PER_TILE)])
  t = c * _NSUB + s
  row0 = t * _NCH2
  pltpu.sync_copy(ei_hbm.at[0, pl.ds(row0, _NCH2)], srcv)
  pltpu.sync_copy(ei_hbm.at[1, pl.ds(row0, _NCH2)], dstv)
  plsc.subcore_barrier()
  _pipeline(hsh, srcv, dstv, acc, buf0, buf1, sem0, sem1, _NCH2)

  @pl.when(t < _TAIL)
  def _():
    _tail_chunk(hsh, ei_hbm, 32 * _NCH2 + t, srcv, dstv, acc, buf0,
                sem0)

  plsc.subcore_barrier()
  pltpu.sync_copy(acc.at[pl.ds(r0, _ROWS_PER_TILE)],
                  out_hbm.at[c, pl.ds(r0, _ROWS_PER_TILE)])


@functools.cache
def _build_aggs():
  mesh = plsc.VectorSubcoreMesh(
      core_axis_name="c", subcore_axis_name="s",
      num_cores=_NCORES, num_subcores=_NSUB)
  params = pltpu.CompilerParams(use_tc_tiling_on_sc=False)
  agg1 = pl.kernel(
      _agg1_body,
      out_type=jax.ShapeDtypeStruct((_NCORES, _NPAD, _DH), jnp.float32),
      mesh=mesh,
      scratch_types=[
          pltpu.VMEM_SHARED((_NPAD, _DH), jnp.float32),
          pltpu.VMEM_SHARED((_NPAD, _DH), jnp.float32),
          pltpu.VMEM((_NCH2, _C), jnp.int32),
          pltpu.VMEM((_NCH2, _C), jnp.int32),
          pltpu.VMEM((_C, _DH), jnp.float32),
          pltpu.VMEM((_C, _DH), jnp.float32),
          pltpu.SemaphoreType.DMA,
          pltpu.SemaphoreType.DMA,
      ],
      compiler_params=params)
  agg2 = pl.kernel(
      _agg2_body,
      out_type=jax.ShapeDtypeStruct((_NCORES, _NPAD, _D2), jnp.float32),
      mesh=mesh,
      scratch_types=[
          pltpu.VMEM_SHARED((_NPAD, _D2), jnp.float32),
          pltpu.VMEM_SHARED((_NPAD, _D2), jnp.float32),
          pltpu.VMEM((_NCH2, _C), jnp.int32),
          pltpu.VMEM((_NCH2, _C), jnp.int32),
          pltpu.VMEM((_C, _D2), jnp.float32),
          pltpu.VMEM((_C, _D2), jnp.float32),
          pltpu.SemaphoreType.DMA,
          pltpu.SemaphoreType.DMA,
      ],
      compiler_params=params)
  return agg1, agg2


def _mm_body(a_ref, w1_ref, w2_ref, o_ref):
  wc = jnp.dot(w1_ref[...], w2_ref[...],
               preferred_element_type=jnp.float32)
  rows = lax.broadcasted_iota(jnp.int32, (8 * _DH, 8 * _D2), 0) // _DH
  cols = lax.broadcasted_iota(jnp.int32, (8 * _DH, 8 * _D2), 1) // _D2
  on_diag = rows == cols
  bda = jnp.where(on_diag, jnp.tile(wc[:_DH], (8, 8)), 0.0)
  bdb = jnp.where(on_diag, jnp.tile(wc[_DH:], (8, 8)), 0.0)
  o_ref[...] = (
      jnp.dot(a_ref[0], bda, preferred_element_type=jnp.float32)
      + jnp.dot(a_ref[1], bdb, preferred_element_type=jnp.float32))


_mm = pl.pallas_call(
    _mm_body,
    out_shape=jax.ShapeDtypeStruct((_NPAD // 8, 8 * _D2), jnp.float32))


def _add_body(a_ref, o_ref):
  o_ref[...] = a_ref[0, :_N // 8] + a_ref[1, :_N // 8]


_add = pl.pallas_call(
    _add_body,
    out_shape=jax.ShapeDtypeStruct((_N // 8, 8 * _D2), jnp.float32))


@jax.jit
def kernel(x, edge_index, W1, W2):
  ei3 = edge_index.astype(jnp.int32).reshape(_NCORES, _ROWS_TOTAL, _C)

  _agg1, _agg2 = _build_aggs()
  zeros1 = jnp.zeros((_NPAD, _DH), jnp.float32)
  agg1 = _agg1(x, ei3, zeros1)

  h2 = _mm(agg1.reshape(_NCORES, _NPAD // 8, 8 * _DH), W1, W2)

  zeros2 = jnp.zeros((_NPAD, _D2), jnp.float32)
  parts = _agg2(h2.reshape(_NPAD, _D2), ei3, zeros2)

  out = _add(parts.reshape(_NCORES, _NPAD // 8, 8 * _D2))
  return out.reshape(_N, _D2)


# speedup vs baseline: 1.2960x; 1.0005x over previous
import functools

import jax
import jax.numpy as jnp
from jax import lax
from jax.experimental import pallas as pl
from jax.experimental.pallas import tpu as pltpu
from jax.experimental.pallas import tpu_sc as plsc

_N = 10000
_NPAD = 10112
_E = 320000
_D1 = 128
_DH = _D1 // 2
_D2 = 16
_C = 128
_NCORES = 2
_NSUB = 16
_ROWS_PER_TILE = _NPAD // _NSUB
_XROWS_PER_TILE = _N // _NSUB

_ROWS_TOTAL = _E // _C
_NCH2 = 78
_PHASES1 = 2
_TAIL = _ROWS_TOTAL - 32 * _NCH2


def _pipeline(table, srcv, dstv, acc, buf0, buf1, sem0, sem1, n_chunks):
  n_pairs = n_chunks // 2
  pltpu.async_copy(table.at[srcv.at[0]], buf0, sem0)

  def body(i, carry):
    j0 = 2 * i
    d1 = pltpu.async_copy(table.at[srcv.at[j0 + 1]], buf1, sem1)
    pltpu.make_async_copy(table.at[srcv.at[j0]], buf0, sem0).wait()
    pltpu.sync_copy(buf0, acc.at[dstv.at[j0]], add=True)
    jpre = jnp.minimum(j0 + 2, n_chunks - 1)
    pltpu.async_copy(table.at[srcv.at[jpre]], buf0, sem0)
    d1.wait()
    pltpu.sync_copy(buf1, acc.at[dstv.at[j0 + 1]], add=True)
    return carry

  lax.fori_loop(0, n_pairs, body, 0)
  pltpu.make_async_copy(table.at[srcv.at[n_chunks - 1]], buf0,
                        sem0).wait()


def _tail_chunk(table, ei_hbm, row, srcv, dstv, acc, buf0, sem0):
  pltpu.sync_copy(ei_hbm.at[0, pl.ds(row, 1)], srcv.at[pl.ds(0, 1)])
  pltpu.sync_copy(ei_hbm.at[1, pl.ds(row, 1)], dstv.at[pl.ds(0, 1)])
  pltpu.async_copy(table.at[srcv.at[0]], buf0, sem0).wait()
  pltpu.sync_copy(buf0, acc.at[dstv.at[0]], add=True)


def _agg1_body(x_hbm, ei_hbm, zeros_hbm, out_hbm,
               xsh, acc, srcv, dstv, buf0, buf1, sem0, sem1):
  c = lax.axis_index("c")
  s = lax.axis_index("s")
  r0 = s * _ROWS_PER_TILE
  pltpu.sync_copy(
      x_hbm.at[pl.ds(s * _XROWS_PER_TILE, _XROWS_PER_TILE),
               pl.ds(c * _DH, _DH)],
      xsh.at[pl.ds(s * _XROWS_PER_TILE, _XROWS_PER_TILE)])
  pltpu.sync_copy(zeros_hbm.at[pl.ds(r0, _ROWS_PER_TILE)],
                  acc.at[pl.ds(r0, _ROWS_PER_TILE)])
  plsc.subcore_barrier()
  for p in range(_PHASES1):
    row0 = s * (_PHASES1 * _NCH2) + p * _NCH2
    pltpu.sync_copy(ei_hbm.at[0, pl.ds(row0, _NCH2)], srcv)
    pltpu.sync_copy(ei_hbm.at[1, pl.ds(row0, _NCH2)], dstv)
    _pipeline(xsh, srcv, dstv, acc, buf0, buf1, sem0, sem1, _NCH2)

  @pl.when(s < _TAIL)
  def _():
    _tail_chunk(xsh, ei_hbm, 32 * _NCH2 + s, srcv, dstv, acc, buf0,
                sem0)

  plsc.subcore_barrier()
  pltpu.sync_copy(acc.at[pl.ds(r0, _ROWS_PER_TILE)],
                  out_hbm.at[c, pl.ds(r0, _ROWS_PER_TILE)])


def _agg2_body(h_hbm, ei_hbm, zeros_hbm, out_hbm,
               hsh, acc, srcv, dstv, buf0, buf1, sem0, sem1):
  c = lax.axis_index("c")
  s = lax.axis_index("s")
  r0 = s * _ROWS_PER_TILE
  pltpu.sync_copy(h_hbm.at[pl.ds(r0, _ROWS_PER_TILE)],
                  hsh.at[pl.ds(r0, _ROWS_PER_TILE)])
  pltpu.sync_copy(zeros_hbm.at[pl.ds(r0, _ROWS_PER_TILE)],
                  acc.at[pl.ds(r0, _ROWS_PER_TILE)])
  t = c * _NSUB + s
  row0 = t * _NCH2
  pltpu.sync_copy(ei_hbm.at[0, pl.ds(row0, _NCH2)], srcv)
  pltpu.sync_copy(ei_hbm.at[1, pl.ds(row0, _NCH2)], dstv)
  plsc.subcore_barrier()
  _pipeline(hsh, srcv, dstv, acc, buf0, buf1, sem0, sem1, _NCH2)

  @pl.when(t < _TAIL)
  def _():
    _tail_chunk(hsh, ei_hbm, 32 * _NCH2 + t, srcv, dstv, acc, buf0,
                sem0)

  plsc.subcore_barrier()
  pltpu.sync_copy(acc.at[pl.ds(r0, _ROWS_PER_TILE)],
                  out_hbm.at[c, pl.ds(r0, _ROWS_PER_TILE)])


@functools.cache
def _build_aggs():
  mesh = plsc.VectorSubcoreMesh(
      core_axis_name="c", subcore_axis_name="s",
      num_cores=_NCORES, num_subcores=_NSUB)
  params = pltpu.CompilerParams(use_tc_tiling_on_sc=False)
  agg1 = pl.kernel(
      _agg1_body,
      out_type=jax.ShapeDtypeStruct((_NCORES, _NPAD, _DH), jnp.float32),
      mesh=mesh,
      scratch_types=[
          pltpu.VMEM_SHARED((_NPAD, _DH), jnp.float32),
          pltpu.VMEM_SHARED((_NPAD, _DH), jnp.float32),
          pltpu.VMEM((_NCH2, _C), jnp.int32),
          pltpu.VMEM((_NCH2, _C), jnp.int32),
          pltpu.VMEM((_C, _DH), jnp.float32),
          pltpu.VMEM((_C, _DH), jnp.float32),
          pltpu.SemaphoreType.DMA,
          pltpu.SemaphoreType.DMA,
      ],
      compiler_params=params)
  agg2 = pl.kernel(
      _agg2_body,
      out_type=jax.ShapeDtypeStruct((_NCORES, _NPAD, _D2), jnp.float32),
      mesh=mesh,
      scratch_types=[
          pltpu.VMEM_SHARED((_NPAD, _D2), jnp.float32),
          pltpu.VMEM_SHARED((_NPAD, _D2), jnp.float32),
          pltpu.VMEM((_NCH2, _C), jnp.int32),
          pltpu.VMEM((_NCH2, _C), jnp.int32),
          pltpu.VMEM((_C, _D2), jnp.float32),
          pltpu.VMEM((_C, _D2), jnp.float32),
          pltpu.SemaphoreType.DMA,
          pltpu.SemaphoreType.DMA,
      ],
      compiler_params=params)
  return agg1, agg2


def _mm_body(a_ref, w1_ref, w2_ref, o_ref):
  wc = jnp.dot(w1_ref[...], w2_ref[...],
               preferred_element_type=jnp.float32)
  rows = lax.broadcasted_iota(jnp.int32, (8 * _DH, 8 * _D2), 0) // _DH
  cols = lax.broadcasted_iota(jnp.int32, (8 * _DH, 8 * _D2), 1) // _D2
  on_diag = rows == cols
  bda = jnp.where(on_diag, jnp.tile(wc[:_DH], (8, 8)), 0.0)
  bdb = jnp.where(on_diag, jnp.tile(wc[_DH:], (8, 8)), 0.0)
  o_ref[...] = (
      jnp.dot(a_ref[0], bda, preferred_element_type=jnp.float32)
      + jnp.dot(a_ref[1], bdb, preferred_element_type=jnp.float32))


_mm = pl.pallas_call(
    _mm_body,
    out_shape=jax.ShapeDtypeStruct((_NPAD // 8, 8 * _D2), jnp.float32))


def _add_body(a_ref, o_ref):
  o_ref[...] = a_ref[0, :_N // 8] + a_ref[1, :_N // 8]


_add = pl.pallas_call(
    _add_body,
    out_shape=jax.ShapeDtypeStruct((_N // 8, 8 * _D2), jnp.float32))


@jax.jit
def kernel(x, edge_index, W1, W2):
  ei3 = edge_index.astype(jnp.int32).reshape(_NCORES, _ROWS_TOTAL, _C)

  _agg1, _agg2 = _build_aggs()
  zeros1 = jnp.zeros((_NPAD, _DH), jnp.float32)
  agg1 = _agg1(x, ei3, zeros1)

  h2 = _mm(agg1.reshape(_NCORES, _NPAD // 8, 8 * _DH), W1, W2)

  zeros2 = jnp.zeros((_NPAD, _D2), jnp.float32)
  parts = _agg2(h2.reshape(_NPAD, _D2), ei3, zeros2)

  out = _add(parts.reshape(_NCORES, _NPAD // 8, 8 * _D2))
  return out.reshape(_N, _D2)
